# Initial kernel scaffold; baseline (speedup 1.0000x reference)
import jax
import jax.numpy as jnp
from jax.experimental import pallas as pl


def kernel(x_adresse, x_batiment, x_parcelle, edge_index_acces, edge_index_desservi, edge_index_appartient, edge_index_contient, edge_attr_acces, edge_attr_desservi, l0_acc_Wl, l0_acc_Wr, l0_acc_bl, l0_acc_br, l0_acc_att, l0_acc_b, l0_acc_We, l0_des_Wl, l0_des_Wr, l0_des_bl, l0_des_br, l0_des_att, l0_des_b, l0_des_We, l0_app_Wl, l0_app_Wr, l0_app_bl, l0_app_br, l0_app_att, l0_app_b, l0_con_Wl, l0_con_Wr, l0_con_bl, l0_con_br, l0_con_att, l0_con_b, l1_acc_Wl, l1_acc_Wr, l1_acc_bl, l1_acc_br, l1_acc_att, l1_acc_b, l1_acc_We, l1_des_Wl, l1_des_Wr, l1_des_bl, l1_des_br, l1_des_att, l1_des_b, l1_des_We, l1_app_Wl, l1_app_Wr, l1_app_bl, l1_app_br, l1_app_att, l1_app_b, l1_con_Wl, l1_con_Wr, l1_con_bl, l1_con_br, l1_con_att, l1_con_b, lin_a_W, lin_a_b, lin_b_W, lin_b_b, lin_p_W, lin_p_b):
    raise NotImplementedError("write your pallas kernel here")


# speedup vs baseline: 5.8058x; 5.8058x over previous
import functools
import jax
import jax.numpy as jnp
from jax import lax
from jax.experimental import pallas as pl
from jax.experimental.pallas import tpu as pltpu
from jax.experimental.pallas import tpu_sc as plsc

N = 10000
E = 320000
D = 128
H = 128
DE = 16
O = 64

NC = 2
NS = 16
NW = NC * NS
EPW = E // NW
B = 80
NCHUNK = EPW // B
AUGW = H + 16


def _mm_kernel(x_ref, w_ref, b_ref, o_ref):
    o_ref[...] = jnp.dot(x_ref[...], w_ref[...],
                         preferred_element_type=jnp.float32) + b_ref[...]


def _matmul_bias(x, w, b, blk):
    m, k = x.shape
    n = w.shape[1]
    return pl.pallas_call(
        _mm_kernel,
        grid=(m // blk,),
        in_specs=[
            pl.BlockSpec((blk, k), lambda i: (i, 0)),
            pl.BlockSpec((k, n), lambda i: (0, 0)),
            pl.BlockSpec((1, n), lambda i: (0, 0)),
        ],
        out_specs=pl.BlockSpec((blk, n), lambda i: (i, 0)),
        out_shape=jax.ShapeDtypeStruct((m, n), jnp.float32),
    )(x, w, b.reshape(1, n))


def _fin1_kernel(n0_ref, n1_ref, d0_ref, d1_ref, b_ref, o_ref):
    num = n0_ref[...] + n1_ref[...]
    den = d0_ref[...] + d1_ref[...]
    o_ref[...] = jnp.maximum(num / (den + 1e-16) + b_ref[...], 0.0)


def _fin2_kernel(na0_ref, na1_ref, da0_ref, da1_ref, ba_ref,
                 nb0_ref, nb1_ref, db0_ref, db1_ref, bb_ref, o_ref):
    xa = (na0_ref[...] + na1_ref[...]) / (da0_ref[...] + da1_ref[...] + 1e-16)
    xb = (nb0_ref[...] + nb1_ref[...]) / (db0_ref[...] + db1_ref[...] + 1e-16)
    o_ref[...] = jnp.maximum(xa + ba_ref[...] + xb + bb_ref[...], 0.0)


def _den_col(den):
    return den.reshape(NC, DROWS * H)[:, :N].reshape(NC, N, 1)


_NSPEC = lambda blk: pl.BlockSpec((blk, H), lambda i: (i, 0))
_DSPEC = lambda blk: pl.BlockSpec((blk, 1), lambda i: (i, 0))
_BSPEC = pl.BlockSpec((1, H), lambda i: (0, 0))


def _finalize1(nd, b, blk=1000):
    num, den = nd
    dc = _den_col(den)
    return pl.pallas_call(
        _fin1_kernel,
        grid=(N // blk,),
        in_specs=[_NSPEC(blk), _NSPEC(blk), _DSPEC(blk), _DSPEC(blk), _BSPEC],
        out_specs=pl.BlockSpec((blk, H), lambda i: (i, 0)),
        out_shape=jax.ShapeDtypeStruct((N, H), jnp.float32),
    )(num[0], num[1], dc[0], dc[1], b.reshape(1, H))


def _finalize2(nda, ba, ndb, bb, blk=1000):
    numa, dena = nda
    numb, denb = ndb
    dca = _den_col(dena)
    dcb = _den_col(denb)
    return pl.pallas_call(
        _fin2_kernel,
        grid=(N // blk,),
        in_specs=[_NSPEC(blk), _NSPEC(blk), _DSPEC(blk), _DSPEC(blk), _BSPEC,
                  _NSPEC(blk), _NSPEC(blk), _DSPEC(blk), _DSPEC(blk), _BSPEC],
        out_specs=pl.BlockSpec((blk, H), lambda i: (i, 0)),
        out_shape=jax.ShapeDtypeStruct((N, H), jnp.float32),
    )(numa[0], numa[1], dca[0], dca[1], ba.reshape(1, H),
      numb[0], numb[1], dcb[0], dcb[1], bb.reshape(1, H))


_GDN = lax.GatherDimensionNumbers(
    offset_dims=(), collapsed_slice_dims=(0,), start_index_map=(0,))

DROWS = 80


def _edge_body(has_ea, *refs):
    if has_ea:
        (xl_hbm, xr_hbm, ea_hbm, att_hbm, src_hbm, dst_hbm, zeros_hbm,
         num_hbm, den_hbm, acc_sh, den_sh, src_v, dst_v, dstp_v, gl_v, gr_v,
         ea_v, att_v, den_v, idx80_v, sem) = refs
    else:
        (xl_hbm, xr_hbm, att_hbm, src_hbm, dst_hbm, zeros_hbm,
         num_hbm, den_hbm, acc_sh, den_sh, src_v, dst_v, dstp_v, gl_v, gr_v,
         ea_v, att_v, den_v, idx80_v, sem) = refs

    cid = lax.axis_index("c")
    sid = lax.axis_index("s")
    wid = cid * NS + sid

    @pl.when(sid == 0)
    def _():
        pltpu.sync_copy(zeros_hbm, acc_sh)
        pltpu.sync_copy(zeros_hbm.at[pl.ds(0, DROWS)], den_sh)

    pltpu.sync_copy(att_hbm, att_v)
    lanes = lax.iota(jnp.int32, 16)
    zero16 = jnp.zeros((16,), jnp.float32)
    for g in range(5):
        idx80_v[pl.ds(g * 16, 16)] = lanes + 16 * g

    def dzero(r, c2):
        for h in range(H // 16):
            den_v[r, pl.ds(h * 16, 16)] = zero16
        return c2

    lax.fori_loop(0, DROWS, dzero, 0)
    plsc.subcore_barrier()

    base = wid * EPW

    def chunk(i, carry):
        off = base + i * B
        pltpu.sync_copy(src_hbm.at[pl.ds(off, B)], src_v)
        pltpu.sync_copy(dst_hbm.at[pl.ds(off, B)], dst_v)
        pltpu.sync_copy(dst_hbm.at[pl.ds(off, B)], dstp_v.at[pl.ds(0, B)])
        cp1 = pltpu.async_copy(xl_hbm.at[src_v], gl_v, sem)
        cp2 = pltpu.async_copy(xr_hbm.at[dst_v], gr_v, sem)
        if has_ea:
            pltpu.sync_copy(ea_hbm.at[pl.ds(off, B)], ea_v)
        cp1.wait()
        cp2.wait()

        def edge(b, c2):
            acc = jnp.zeros((16,), jnp.float32)
            for h in range(H // 16):
                sl = pl.ds(h * 16, 16)
                v = gl_v[b, sl] + gr_v[b, sl]
                if has_ea:
                    v = v + ea_v[b, sl]
                v = jnp.where(v > 0.0, v, 0.2 * v)
                acc = acc + v * att_v[sl]
            for k in (8, 4, 2, 1):
                acc = acc + lax.gather(
                    acc, jnp.reshape(lanes ^ k, (16, 1)), _GDN, (1,),
                    mode=lax.GatherScatterMode.PROMISE_IN_BOUNDS)
            w = jnp.exp(acc)
            for h in range(H // 16):
                sl = pl.ds(h * 16, 16)
                gl_v[b, sl] = gl_v[b, sl] * w
            dvec = dstp_v[pl.ds(b, 16)]
            d = dvec[0]
            dr = lax.shift_right_logical(d, 7)
            col0 = pl.multiple_of(lax.bitwise_and(d, 112), 16)
            lane = lax.bitwise_and(d, 15)
            sl = pl.ds(col0, 16)
            den_v[dr, sl] = den_v[dr, sl] + jnp.where(lanes == lane, w, 0.0)
            return c2

        lax.fori_loop(0, B, edge, 0)
        pltpu.sync_copy(gl_v, acc_sh.at[dst_v], add=True)
        return carry

    lax.fori_loop(0, NCHUNK, chunk, 0)
    pltpu.sync_copy(den_v, den_sh.at[idx80_v], add=True)
    plsc.subcore_barrier()

    @pl.when(sid == 0)
    def _():
        pltpu.sync_copy(acc_sh, num_hbm.at[cid])
        pltpu.sync_copy(den_sh, den_hbm.at[cid])


def _make_edge_pass(has_ea):
    mesh = plsc.VectorSubcoreMesh(core_axis_name="c", subcore_axis_name="s")
    return pl.kernel(
        functools.partial(_edge_body, has_ea),
        out_type=(jax.ShapeDtypeStruct((NC, N, H), jnp.float32),
                  jax.ShapeDtypeStruct((NC, DROWS, H), jnp.float32)),
        mesh=mesh,
        scratch_types=[
            pltpu.VMEM_SHARED((N, H), jnp.float32),
            pltpu.VMEM_SHARED((DROWS, H), jnp.float32),
            pltpu.VMEM((B,), jnp.int32),
            pltpu.VMEM((B,), jnp.int32),
            pltpu.VMEM((B + 16,), jnp.int32),
            pltpu.VMEM((B, H), jnp.float32),
            pltpu.VMEM((B, H), jnp.float32),
            pltpu.VMEM((B, H), jnp.float32),
            pltpu.VMEM((H,), jnp.float32),
            pltpu.VMEM((DROWS, H), jnp.float32),
            pltpu.VMEM((DROWS,), jnp.int32),
            pltpu.SemaphoreType.DMA,
        ],
    )


_edge_pass_ea = _make_edge_pass(True)
_edge_pass_noea = _make_edge_pass(False)


def _gatv2(x_src, x_dst, ei, p, zeros, edge_attr=None):
    xl = _matmul_bias(x_src, p["Wl"], p["bl"], blk=1000)
    xr = _matmul_bias(x_dst, p["Wr"], p["br"], blk=1000)
    src, dst = ei[0], ei[1]
    if edge_attr is not None:
        ea = _matmul_bias(edge_attr, p["We"], jnp.zeros((H,), jnp.float32),
                          blk=2000)
        aug = _edge_pass_ea(xl, xr, ea, p["att"], src, dst, zeros)
    else:
        aug = _edge_pass_noea(xl, xr, p["att"], src, dst, zeros)
    return aug


def kernel(x_adresse, x_batiment, x_parcelle, edge_index_acces, edge_index_desservi, edge_index_appartient, edge_index_contient, edge_attr_acces, edge_attr_desservi, l0_acc_Wl, l0_acc_Wr, l0_acc_bl, l0_acc_br, l0_acc_att, l0_acc_b, l0_acc_We, l0_des_Wl, l0_des_Wr, l0_des_bl, l0_des_br, l0_des_att, l0_des_b, l0_des_We, l0_app_Wl, l0_app_Wr, l0_app_bl, l0_app_br, l0_app_att, l0_app_b, l0_con_Wl, l0_con_Wr, l0_con_bl, l0_con_br, l0_con_att, l0_con_b, l1_acc_Wl, l1_acc_Wr, l1_acc_bl, l1_acc_br, l1_acc_att, l1_acc_b, l1_acc_We, l1_des_Wl, l1_des_Wr, l1_des_bl, l1_des_br, l1_des_att, l1_des_b, l1_des_We, l1_app_Wl, l1_app_Wr, l1_app_bl, l1_app_br, l1_app_att, l1_app_b, l1_con_Wl, l1_con_Wr, l1_con_bl, l1_con_br, l1_con_att, l1_con_b, lin_a_W, lin_a_b, lin_b_W, lin_b_b, lin_p_W, lin_p_b):
    inp = dict(locals())
    zeros = jnp.zeros((N, H), jnp.float32)

    def prm(pre):
        keys = ["Wl", "Wr", "bl", "br", "att", "b"]
        p = {k: inp[pre + k] for k in keys}
        if (pre + "We") in inp:
            p["We"] = inp[pre + "We"]
        return p

    xa, xb, xp = x_adresse, x_batiment, x_parcelle
    for l in range(2):
        pre = "l%d_" % l
        nd_acc = _gatv2(xa, xb, edge_index_acces, prm(pre + "acc_"),
                        zeros, edge_attr_acces)
        nd_con = _gatv2(xp, xb, edge_index_contient, prm(pre + "con_"),
                        zeros)
        nd_des = _gatv2(xb, xa, edge_index_desservi, prm(pre + "des_"),
                        zeros, edge_attr_desservi)
        nd_app = _gatv2(xb, xp, edge_index_appartient, prm(pre + "app_"),
                        zeros)
        xb_new = _finalize2(nd_acc, inp[pre + "acc_b"],
                            nd_con, inp[pre + "con_b"])
        xa = _finalize1(nd_des, inp[pre + "des_b"])
        xp = _finalize1(nd_app, inp[pre + "app_b"])
        xb = xb_new

    return (_matmul_bias(xa, lin_a_W, lin_a_b, blk=1000),
            _matmul_bias(xb, lin_b_W, lin_b_b, blk=1000),
            _matmul_bias(xp, lin_p_W, lin_p_b, blk=1000))


# speedup vs baseline: 6.0486x; 1.0418x over previous
import functools
import jax
import jax.numpy as jnp
from jax import lax
from jax.experimental import pallas as pl
from jax.experimental.pallas import tpu as pltpu
from jax.experimental.pallas import tpu_sc as plsc

N = 10000
E = 320000
D = 128
H = 128
DE = 16
O = 64

NC = 2
NS = 16
NW = NC * NS
EPW = E // NW
B = 40
NCHUNK = EPW // B
AUGW = H + 16


def _mm_kernel(x_ref, w_ref, b_ref, o_ref):
    o_ref[...] = jnp.dot(x_ref[...], w_ref[...],
                         preferred_element_type=jnp.float32) + b_ref[...]


def _matmul_bias(x, w, b, blk):
    m, k = x.shape
    n = w.shape[1]
    return pl.pallas_call(
        _mm_kernel,
        grid=(m // blk,),
        in_specs=[
            pl.BlockSpec((blk, k), lambda i: (i, 0)),
            pl.BlockSpec((k, n), lambda i: (0, 0)),
            pl.BlockSpec((1, n), lambda i: (0, 0)),
        ],
        out_specs=pl.BlockSpec((blk, n), lambda i: (i, 0)),
        out_shape=jax.ShapeDtypeStruct((m, n), jnp.float32),
    )(x, w, b.reshape(1, n))


def _fin1_kernel(n0_ref, n1_ref, d0_ref, d1_ref, b_ref, o_ref):
    num = n0_ref[...] + n1_ref[...]
    den = d0_ref[...] + d1_ref[...]
    o_ref[...] = jnp.maximum(num / (den + 1e-16) + b_ref[...], 0.0)


def _fin2_kernel(na0_ref, na1_ref, da0_ref, da1_ref, ba_ref,
                 nb0_ref, nb1_ref, db0_ref, db1_ref, bb_ref, o_ref):
    xa = (na0_ref[...] + na1_ref[...]) / (da0_ref[...] + da1_ref[...] + 1e-16)
    xb = (nb0_ref[...] + nb1_ref[...]) / (db0_ref[...] + db1_ref[...] + 1e-16)
    o_ref[...] = jnp.maximum(xa + ba_ref[...] + xb + bb_ref[...], 0.0)


def _den_col(den):
    return den.reshape(NC, DROWS * H)[:, :N].reshape(NC, N, 1)


_NSPEC = lambda blk: pl.BlockSpec((blk, H), lambda i: (i, 0))
_DSPEC = lambda blk: pl.BlockSpec((blk, 1), lambda i: (i, 0))
_BSPEC = pl.BlockSpec((1, H), lambda i: (0, 0))


def _finalize1(nd, b, blk=1000):
    num, den = nd
    dc = _den_col(den)
    return pl.pallas_call(
        _fin1_kernel,
        grid=(N // blk,),
        in_specs=[_NSPEC(blk), _NSPEC(blk), _DSPEC(blk), _DSPEC(blk), _BSPEC],
        out_specs=pl.BlockSpec((blk, H), lambda i: (i, 0)),
        out_shape=jax.ShapeDtypeStruct((N, H), jnp.float32),
    )(num[0], num[1], dc[0], dc[1], b.reshape(1, H))


def _finalize2(nda, ba, ndb, bb, blk=1000):
    numa, dena = nda
    numb, denb = ndb
    dca = _den_col(dena)
    dcb = _den_col(denb)
    return pl.pallas_call(
        _fin2_kernel,
        grid=(N // blk,),
        in_specs=[_NSPEC(blk), _NSPEC(blk), _DSPEC(blk), _DSPEC(blk), _BSPEC,
                  _NSPEC(blk), _NSPEC(blk), _DSPEC(blk), _DSPEC(blk), _BSPEC],
        out_specs=pl.BlockSpec((blk, H), lambda i: (i, 0)),
        out_shape=jax.ShapeDtypeStruct((N, H), jnp.float32),
    )(numa[0], numa[1], dca[0], dca[1], ba.reshape(1, H),
      numb[0], numb[1], dcb[0], dcb[1], bb.reshape(1, H))


_GDN = lax.GatherDimensionNumbers(
    offset_dims=(), collapsed_slice_dims=(0,), start_index_map=(0,))

DROWS = 80


def _edge_body(has_ea, *refs):
    if has_ea:
        (xl_hbm, xr_hbm, ea_hbm, att_hbm, src_hbm, dst_hbm, zeros_hbm,
         num_hbm, den_hbm, acc_sh, den_sh,
         src0_v, dst0_v, dstp0_v, gl0_v, gr0_v, ea0_v,
         src1_v, dst1_v, dstp1_v, gl1_v, gr1_v, ea1_v,
         att_v, den_v, idx80_v, sem0, sem1) = refs
    else:
        ea_hbm = None
        (xl_hbm, xr_hbm, att_hbm, src_hbm, dst_hbm, zeros_hbm,
         num_hbm, den_hbm, acc_sh, den_sh,
         src0_v, dst0_v, dstp0_v, gl0_v, gr0_v, ea0_v,
         src1_v, dst1_v, dstp1_v, gl1_v, gr1_v, ea1_v,
         att_v, den_v, idx80_v, sem0, sem1) = refs
    bufs = ((src0_v, dst0_v, dstp0_v, gl0_v, gr0_v, ea0_v, sem0),
            (src1_v, dst1_v, dstp1_v, gl1_v, gr1_v, ea1_v, sem1))

    cid = lax.axis_index("c")
    sid = lax.axis_index("s")
    wid = cid * NS + sid

    @pl.when(sid == 0)
    def _():
        pltpu.sync_copy(zeros_hbm, acc_sh)
        pltpu.sync_copy(zeros_hbm.at[pl.ds(0, DROWS)], den_sh)

    pltpu.sync_copy(att_hbm, att_v)
    lanes = lax.iota(jnp.int32, 16)
    zero16 = jnp.zeros((16,), jnp.float32)
    for g in range(5):
        idx80_v[pl.ds(g * 16, 16)] = lanes + 16 * g

    def dzero(r, c2):
        for h in range(H // 16):
            den_v[r, pl.ds(h * 16, 16)] = zero16
        return c2

    lax.fori_loop(0, DROWS, dzero, 0)
    plsc.subcore_barrier()

    base = wid * EPW

    def issue_load(i, b):
        src_v, dst_v, dstp_v, gl_v, gr_v, ea_v, sem = bufs[b]
        off = base + i * B
        pltpu.sync_copy(src_hbm.at[pl.ds(off, B)], src_v)
        pltpu.sync_copy(dst_hbm.at[pl.ds(off, B)], dst_v)
        pltpu.sync_copy(dst_hbm.at[pl.ds(off, B)], dstp_v.at[pl.ds(0, B)])
        pltpu.async_copy(xl_hbm.at[src_v], gl_v, sem)
        pltpu.async_copy(xr_hbm.at[dst_v], gr_v, sem)
        if has_ea:
            pltpu.async_copy(ea_hbm.at[pl.ds(off, B)], ea_v, sem)

    def wait_load(b):
        src_v, dst_v, dstp_v, gl_v, gr_v, ea_v, sem = bufs[b]
        pltpu.make_async_copy(xl_hbm.at[src_v], gl_v, sem).wait()
        pltpu.make_async_copy(xr_hbm.at[dst_v], gr_v, sem).wait()
        if has_ea:
            pltpu.make_async_copy(ea_hbm.at[pl.ds(0, B)], ea_v, sem).wait()

    def compute(bi):
        src_v, dst_v, dstp_v, gl_v, gr_v, ea_v, sem = bufs[bi]

        def edge(b, c2):
            acc = jnp.zeros((16,), jnp.float32)
            for h in range(H // 16):
                sl = pl.ds(h * 16, 16)
                v = gl_v[b, sl] + gr_v[b, sl]
                if has_ea:
                    v = v + ea_v[b, sl]
                v = jnp.where(v > 0.0, v, 0.2 * v)
                acc = acc + v * att_v[sl]
            for k in (8, 4, 2, 1):
                acc = acc + lax.gather(
                    acc, jnp.reshape(lanes ^ k, (16, 1)), _GDN, (1,),
                    mode=lax.GatherScatterMode.PROMISE_IN_BOUNDS)
            w = jnp.exp(acc)
            for h in range(H // 16):
                sl = pl.ds(h * 16, 16)
                gl_v[b, sl] = gl_v[b, sl] * w
            dvec = dstp_v[pl.ds(b, 16)]
            d = dvec[0]
            dr = lax.shift_right_logical(d, 7)
            col0 = pl.multiple_of(lax.bitwise_and(d, 112), 16)
            lane = lax.bitwise_and(d, 15)
            sl = pl.ds(col0, 16)
            den_v[dr, sl] = den_v[dr, sl] + jnp.where(lanes == lane, w, 0.0)
            return c2

        lax.fori_loop(0, B, edge, 0)
        pltpu.sync_copy(gl_v, acc_sh.at[dst_v], add=True)

    issue_load(0, 0)

    def pair(j, carry):
        for bi in range(2):
            i = 2 * j + bi

            @pl.when(i + 1 < NCHUNK)
            def _(bi=bi, i=i):
                issue_load(i + 1, 1 - bi)

            @pl.when(i < NCHUNK)
            def _(bi=bi, i=i):
                wait_load(bi)
                compute(bi)
        return carry

    lax.fori_loop(0, (NCHUNK + 1) // 2, pair, 0)
    pltpu.sync_copy(den_v, den_sh.at[idx80_v], add=True)
    plsc.subcore_barrier()

    @pl.when(sid == 0)
    def _():
        pltpu.sync_copy(acc_sh, num_hbm.at[cid])
        pltpu.sync_copy(den_sh, den_hbm.at[cid])


def _make_edge_pass(has_ea):
    mesh = plsc.VectorSubcoreMesh(core_axis_name="c", subcore_axis_name="s")
    return pl.kernel(
        functools.partial(_edge_body, has_ea),
        out_type=(jax.ShapeDtypeStruct((NC, N, H), jnp.float32),
                  jax.ShapeDtypeStruct((NC, DROWS, H), jnp.float32)),
        mesh=mesh,
        scratch_types=[
            pltpu.VMEM_SHARED((N, H), jnp.float32),
            pltpu.VMEM_SHARED((DROWS, H), jnp.float32),
            pltpu.VMEM((B,), jnp.int32),
            pltpu.VMEM((B,), jnp.int32),
            pltpu.VMEM((B + 16,), jnp.int32),
            pltpu.VMEM((B, H), jnp.float32),
            pltpu.VMEM((B, H), jnp.float32),
            pltpu.VMEM((B, H), jnp.float32),
            pltpu.VMEM((B,), jnp.int32),
            pltpu.VMEM((B,), jnp.int32),
            pltpu.VMEM((B + 16,), jnp.int32),
            pltpu.VMEM((B, H), jnp.float32),
            pltpu.VMEM((B, H), jnp.float32),
            pltpu.VMEM((B, H), jnp.float32),
            pltpu.VMEM((H,), jnp.float32),
            pltpu.VMEM((DROWS, H), jnp.float32),
            pltpu.VMEM((DROWS,), jnp.int32),
            pltpu.SemaphoreType.DMA,
            pltpu.SemaphoreType.DMA,
        ],
    )


_edge_pass_ea = _make_edge_pass(True)
_edge_pass_noea = _make_edge_pass(False)


def _gatv2(x_src, x_dst, ei, p, zeros, edge_attr=None):
    xl = _matmul_bias(x_src, p["Wl"], p["bl"], blk=1000)
    xr = _matmul_bias(x_dst, p["Wr"], p["br"], blk=1000)
    src, dst = ei[0], ei[1]
    if edge_attr is not None:
        ea = _matmul_bias(edge_attr, p["We"], jnp.zeros((H,), jnp.float32),
                          blk=2000)
        aug = _edge_pass_ea(xl, xr, ea, p["att"], src, dst, zeros)
    else:
        aug = _edge_pass_noea(xl, xr, p["att"], src, dst, zeros)
    return aug


def kernel(x_adresse, x_batiment, x_parcelle, edge_index_acces, edge_index_desservi, edge_index_appartient, edge_index_contient, edge_attr_acces, edge_attr_desservi, l0_acc_Wl, l0_acc_Wr, l0_acc_bl, l0_acc_br, l0_acc_att, l0_acc_b, l0_acc_We, l0_des_Wl, l0_des_Wr, l0_des_bl, l0_des_br, l0_des_att, l0_des_b, l0_des_We, l0_app_Wl, l0_app_Wr, l0_app_bl, l0_app_br, l0_app_att, l0_app_b, l0_con_Wl, l0_con_Wr, l0_con_bl, l0_con_br, l0_con_att, l0_con_b, l1_acc_Wl, l1_acc_Wr, l1_acc_bl, l1_acc_br, l1_acc_att, l1_acc_b, l1_acc_We, l1_des_Wl, l1_des_Wr, l1_des_bl, l1_des_br, l1_des_att, l1_des_b, l1_des_We, l1_app_Wl, l1_app_Wr, l1_app_bl, l1_app_br, l1_app_att, l1_app_b, l1_con_Wl, l1_con_Wr, l1_con_bl, l1_con_br, l1_con_att, l1_con_b, lin_a_W, lin_a_b, lin_b_W, lin_b_b, lin_p_W, lin_p_b):
    inp = dict(locals())
    zeros = jnp.zeros((N, H), jnp.float32)

    def prm(pre):
        keys = ["Wl", "Wr", "bl", "br", "att", "b"]
        p = {k: inp[pre + k] for k in keys}
        if (pre + "We") in inp:
            p["We"] = inp[pre + "We"]
        return p

    xa, xb, xp = x_adresse, x_batiment, x_parcelle
    for l in range(2):
        pre = "l%d_" % l
        nd_acc = _gatv2(xa, xb, edge_index_acces, prm(pre + "acc_"),
                        zeros, edge_attr_acces)
        nd_con = _gatv2(xp, xb, edge_index_contient, prm(pre + "con_"),
                        zeros)
        nd_des = _gatv2(xb, xa, edge_index_desservi, prm(pre + "des_"),
                        zeros, edge_attr_desservi)
        nd_app = _gatv2(xb, xp, edge_index_appartient, prm(pre + "app_"),
                        zeros)
        xb_new = _finalize2(nd_acc, inp[pre + "acc_b"],
                            nd_con, inp[pre + "con_b"])
        xa = _finalize1(nd_des, inp[pre + "des_b"])
        xp = _finalize1(nd_app, inp[pre + "app_b"])
        xb = xb_new

    return (_matmul_bias(xa, lin_a_W, lin_a_b, blk=1000),
            _matmul_bias(xb, lin_b_W, lin_b_b, blk=1000),
            _matmul_bias(xp, lin_p_W, lin_p_b, blk=1000))


# speedup vs baseline: 6.2131x; 1.0272x over previous
import functools
import jax
import jax.numpy as jnp
from jax import lax
from jax.experimental import pallas as pl
from jax.experimental.pallas import tpu as pltpu
from jax.experimental.pallas import tpu_sc as plsc

N = 10000
E = 320000
D = 128
H = 128
DE = 16
O = 64

NC = 2
NS = 16
NW = NC * NS
EPW = E // NW
B = 40
NCHUNK = EPW // B
AUGW = H + 16


def _mm_kernel(x_ref, w_ref, b_ref, o_ref):
    o_ref[...] = jnp.dot(x_ref[...], w_ref[...],
                         preferred_element_type=jnp.float32) + b_ref[...]


def _matmul_bias(x, w, b, blk):
    m, k = x.shape
    n = w.shape[1]
    return pl.pallas_call(
        _mm_kernel,
        grid=(m // blk,),
        in_specs=[
            pl.BlockSpec((blk, k), lambda i: (i, 0)),
            pl.BlockSpec((k, n), lambda i: (0, 0)),
            pl.BlockSpec((1, n), lambda i: (0, 0)),
        ],
        out_specs=pl.BlockSpec((blk, n), lambda i: (i, 0)),
        out_shape=jax.ShapeDtypeStruct((m, n), jnp.float32),
    )(x, w, b.reshape(1, n))


def _fin1_kernel(n0_ref, n1_ref, d0_ref, d1_ref, b_ref, o_ref):
    num = n0_ref[...] + n1_ref[...]
    den = d0_ref[...] + d1_ref[...]
    o_ref[...] = jnp.maximum(num / (den + 1e-16) + b_ref[...], 0.0)


def _fin2_kernel(na0_ref, na1_ref, da0_ref, da1_ref, ba_ref,
                 nb0_ref, nb1_ref, db0_ref, db1_ref, bb_ref, o_ref):
    xa = (na0_ref[...] + na1_ref[...]) / (da0_ref[...] + da1_ref[...] + 1e-16)
    xb = (nb0_ref[...] + nb1_ref[...]) / (db0_ref[...] + db1_ref[...] + 1e-16)
    o_ref[...] = jnp.maximum(xa + ba_ref[...] + xb + bb_ref[...], 0.0)


def _den_col(den):
    return den.reshape(NC, DROWS * H)[:, :N].reshape(NC, N, 1)


_NSPEC = lambda blk: pl.BlockSpec((blk, H), lambda i: (i, 0))
_DSPEC = lambda blk: pl.BlockSpec((blk, 1), lambda i: (i, 0))
_BSPEC = pl.BlockSpec((1, H), lambda i: (0, 0))


def _finalize1(nd, b, blk=1000):
    num, den = nd
    dc = _den_col(den)
    return pl.pallas_call(
        _fin1_kernel,
        grid=(N // blk,),
        in_specs=[_NSPEC(blk), _NSPEC(blk), _DSPEC(blk), _DSPEC(blk), _BSPEC],
        out_specs=pl.BlockSpec((blk, H), lambda i: (i, 0)),
        out_shape=jax.ShapeDtypeStruct((N, H), jnp.float32),
    )(num[0], num[1], dc[0], dc[1], b.reshape(1, H))


def _finalize2(nda, ba, ndb, bb, blk=1000):
    numa, dena = nda
    numb, denb = ndb
    dca = _den_col(dena)
    dcb = _den_col(denb)
    return pl.pallas_call(
        _fin2_kernel,
        grid=(N // blk,),
        in_specs=[_NSPEC(blk), _NSPEC(blk), _DSPEC(blk), _DSPEC(blk), _BSPEC,
                  _NSPEC(blk), _NSPEC(blk), _DSPEC(blk), _DSPEC(blk), _BSPEC],
        out_specs=pl.BlockSpec((blk, H), lambda i: (i, 0)),
        out_shape=jax.ShapeDtypeStruct((N, H), jnp.float32),
    )(numa[0], numa[1], dca[0], dca[1], ba.reshape(1, H),
      numb[0], numb[1], dcb[0], dcb[1], bb.reshape(1, H))


_GDN = lax.GatherDimensionNumbers(
    offset_dims=(), collapsed_slice_dims=(0,), start_index_map=(0,))

DROWS = 80


def _edge_body(has_ea, *refs):
    if has_ea:
        (xl_hbm, xr_hbm, ea_hbm, att_hbm, src_hbm, dst_hbm, zeros_hbm,
         num_hbm, den_hbm, acc_sh, den_sh,
         src0_v, dst0_v, dstp0_v, gl0_v, gr0_v, ea0_v,
         src1_v, dst1_v, dstp1_v, gl1_v, gr1_v, ea1_v,
         att_v, den_v, idx80_v, sem0, sem1) = refs
    else:
        ea_hbm = None
        (xl_hbm, xr_hbm, att_hbm, src_hbm, dst_hbm, zeros_hbm,
         num_hbm, den_hbm, acc_sh, den_sh,
         src0_v, dst0_v, dstp0_v, gl0_v, gr0_v, ea0_v,
         src1_v, dst1_v, dstp1_v, gl1_v, gr1_v, ea1_v,
         att_v, den_v, idx80_v, sem0, sem1) = refs
    bufs = ((src0_v, dst0_v, dstp0_v, gl0_v, gr0_v, ea0_v, sem0),
            (src1_v, dst1_v, dstp1_v, gl1_v, gr1_v, ea1_v, sem1))

    cid = lax.axis_index("c")
    sid = lax.axis_index("s")
    wid = cid * NS + sid

    @pl.when(sid == 0)
    def _():
        pltpu.sync_copy(zeros_hbm, acc_sh)
        pltpu.sync_copy(zeros_hbm.at[pl.ds(0, DROWS)], den_sh)

    pltpu.sync_copy(att_hbm, att_v)
    lanes = lax.iota(jnp.int32, 16)
    zero16 = jnp.zeros((16,), jnp.float32)
    for g in range(5):
        idx80_v[pl.ds(g * 16, 16)] = lanes + 16 * g

    def dzero(r, c2):
        for h in range(H // 16):
            den_v[r, pl.ds(h * 16, 16)] = zero16
        return c2

    lax.fori_loop(0, DROWS, dzero, 0)
    plsc.subcore_barrier()

    base = wid * EPW
    atts = tuple(att_v[pl.ds(h * 16, 16)] for h in range(H // 16))

    def issue_load(i, b):
        src_v, dst_v, dstp_v, gl_v, gr_v, ea_v, sem = bufs[b]
        off = base + i * B
        pltpu.sync_copy(src_hbm.at[pl.ds(off, B)], src_v)
        pltpu.sync_copy(dst_hbm.at[pl.ds(off, B)], dst_v)
        pltpu.sync_copy(dst_hbm.at[pl.ds(off, B)], dstp_v.at[pl.ds(0, B)])
        pltpu.async_copy(xl_hbm.at[src_v], gl_v, sem)
        pltpu.async_copy(xr_hbm.at[dst_v], gr_v, sem)
        if has_ea:
            pltpu.async_copy(ea_hbm.at[pl.ds(off, B)], ea_v, sem)

    def wait_load(b):
        src_v, dst_v, dstp_v, gl_v, gr_v, ea_v, sem = bufs[b]
        pltpu.make_async_copy(xl_hbm.at[src_v], gl_v, sem).wait()
        pltpu.make_async_copy(xr_hbm.at[dst_v], gr_v, sem).wait()
        if has_ea:
            pltpu.make_async_copy(ea_hbm.at[pl.ds(0, B)], ea_v, sem).wait()

    def compute(bi):
        src_v, dst_v, dstp_v, gl_v, gr_v, ea_v, sem = bufs[bi]

        def edge(b, c2):
            acc = jnp.zeros((16,), jnp.float32)
            for h in range(H // 16):
                sl = pl.ds(h * 16, 16)
                v = gl_v[b, sl] + gr_v[b, sl]
                if has_ea:
                    v = v + ea_v[b, sl]
                v = jnp.where(v > 0.0, v, 0.2 * v)
                acc = acc + v * atts[h]
            for k in (8, 4, 2, 1):
                acc = acc + lax.gather(
                    acc, jnp.reshape(lanes ^ k, (16, 1)), _GDN, (1,),
                    mode=lax.GatherScatterMode.PROMISE_IN_BOUNDS)
            w = jnp.exp(acc)
            for h in range(H // 16):
                sl = pl.ds(h * 16, 16)
                gl_v[b, sl] = gl_v[b, sl] * w
            dvec = dstp_v[pl.ds(b, 16)]
            d = dvec[0]
            dr = lax.shift_right_logical(d, 7)
            col0 = pl.multiple_of(lax.bitwise_and(d, 112), 16)
            lane = lax.bitwise_and(d, 15)
            sl = pl.ds(col0, 16)
            den_v[dr, sl] = den_v[dr, sl] + jnp.where(lanes == lane, w, 0.0)
            return c2

        lax.fori_loop(0, B, edge, 0, unroll=4)
        pltpu.sync_copy(gl_v, acc_sh.at[dst_v], add=True)

    issue_load(0, 0)

    def pair(j, carry):
        for bi in range(2):
            i = 2 * j + bi

            @pl.when(i + 1 < NCHUNK)
            def _(bi=bi, i=i):
                issue_load(i + 1, 1 - bi)

            @pl.when(i < NCHUNK)
            def _(bi=bi, i=i):
                wait_load(bi)
                compute(bi)
        return carry

    lax.fori_loop(0, (NCHUNK + 1) // 2, pair, 0)
    pltpu.sync_copy(den_v, den_sh.at[idx80_v], add=True)
    plsc.subcore_barrier()

    @pl.when(sid == 0)
    def _():
        pltpu.sync_copy(acc_sh, num_hbm.at[cid])
        pltpu.sync_copy(den_sh, den_hbm.at[cid])


def _make_edge_pass(has_ea):
    mesh = plsc.VectorSubcoreMesh(core_axis_name="c", subcore_axis_name="s")
    return pl.kernel(
        functools.partial(_edge_body, has_ea),
        out_type=(jax.ShapeDtypeStruct((NC, N, H), jnp.float32),
                  jax.ShapeDtypeStruct((NC, DROWS, H), jnp.float32)),
        mesh=mesh,
        scratch_types=[
            pltpu.VMEM_SHARED((N, H), jnp.float32),
            pltpu.VMEM_SHARED((DROWS, H), jnp.float32),
            pltpu.VMEM((B,), jnp.int32),
            pltpu.VMEM((B,), jnp.int32),
            pltpu.VMEM((B + 16,), jnp.int32),
            pltpu.VMEM((B, H), jnp.float32),
            pltpu.VMEM((B, H), jnp.float32),
            pltpu.VMEM((B, H), jnp.float32),
            pltpu.VMEM((B,), jnp.int32),
            pltpu.VMEM((B,), jnp.int32),
            pltpu.VMEM((B + 16,), jnp.int32),
            pltpu.VMEM((B, H), jnp.float32),
            pltpu.VMEM((B, H), jnp.float32),
            pltpu.VMEM((B, H), jnp.float32),
            pltpu.VMEM((H,), jnp.float32),
            pltpu.VMEM((DROWS, H), jnp.float32),
            pltpu.VMEM((DROWS,), jnp.int32),
            pltpu.SemaphoreType.DMA,
            pltpu.SemaphoreType.DMA,
        ],
    )


_edge_pass_ea = _make_edge_pass(True)
_edge_pass_noea = _make_edge_pass(False)


def _gatv2(x_src, x_dst, ei, p, zeros, edge_attr=None):
    xl = _matmul_bias(x_src, p["Wl"], p["bl"], blk=1000)
    xr = _matmul_bias(x_dst, p["Wr"], p["br"], blk=1000)
    src, dst = ei[0], ei[1]
    if edge_attr is not None:
        ea = _matmul_bias(edge_attr, p["We"], jnp.zeros((H,), jnp.float32),
                          blk=2000)
        aug = _edge_pass_ea(xl, xr, ea, p["att"], src, dst, zeros)
    else:
        aug = _edge_pass_noea(xl, xr, p["att"], src, dst, zeros)
    return aug


def kernel(x_adresse, x_batiment, x_parcelle, edge_index_acces, edge_index_desservi, edge_index_appartient, edge_index_contient, edge_attr_acces, edge_attr_desservi, l0_acc_Wl, l0_acc_Wr, l0_acc_bl, l0_acc_br, l0_acc_att, l0_acc_b, l0_acc_We, l0_des_Wl, l0_des_Wr, l0_des_bl, l0_des_br, l0_des_att, l0_des_b, l0_des_We, l0_app_Wl, l0_app_Wr, l0_app_bl, l0_app_br, l0_app_att, l0_app_b, l0_con_Wl, l0_con_Wr, l0_con_bl, l0_con_br, l0_con_att, l0_con_b, l1_acc_Wl, l1_acc_Wr, l1_acc_bl, l1_acc_br, l1_acc_att, l1_acc_b, l1_acc_We, l1_des_Wl, l1_des_Wr, l1_des_bl, l1_des_br, l1_des_att, l1_des_b, l1_des_We, l1_app_Wl, l1_app_Wr, l1_app_bl, l1_app_br, l1_app_att, l1_app_b, l1_con_Wl, l1_con_Wr, l1_con_bl, l1_con_br, l1_con_att, l1_con_b, lin_a_W, lin_a_b, lin_b_W, lin_b_b, lin_p_W, lin_p_b):
    inp = dict(locals())
    zeros = jnp.zeros((N, H), jnp.float32)

    def prm(pre):
        keys = ["Wl", "Wr", "bl", "br", "att", "b"]
        p = {k: inp[pre + k] for k in keys}
        if (pre + "We") in inp:
            p["We"] = inp[pre + "We"]
        return p

    xa, xb, xp = x_adresse, x_batiment, x_parcelle
    for l in range(2):
        pre = "l%d_" % l
        nd_acc = _gatv2(xa, xb, edge_index_acces, prm(pre + "acc_"),
                        zeros, edge_attr_acces)
        nd_con = _gatv2(xp, xb, edge_index_contient, prm(pre + "con_"),
                        zeros)
        nd_des = _gatv2(xb, xa, edge_index_desservi, prm(pre + "des_"),
                        zeros, edge_attr_desservi)
        nd_app = _gatv2(xb, xp, edge_index_appartient, prm(pre + "app_"),
                        zeros)
        xb_new = _finalize2(nd_acc, inp[pre + "acc_b"],
                            nd_con, inp[pre + "con_b"])
        xa = _finalize1(nd_des, inp[pre + "des_b"])
        xp = _finalize1(nd_app, inp[pre + "app_b"])
        xb = xb_new

    return (_matmul_bias(xa, lin_a_W, lin_a_b, blk=1000),
            _matmul_bias(xb, lin_b_W, lin_b_b, blk=1000),
            _matmul_bias(xp, lin_p_W, lin_p_b, blk=1000))
